# baseline (speedup 1.0000x reference)
import functools

import jax
import jax.numpy as jnp
from jax import lax
from jax.experimental import pallas as pl
from jax.experimental.pallas import tpu as pltpu
from jax.experimental.pallas import tpu_sc as plsc

B = 16384
C = 5
E = 64
V = 1000000
NC, NS = 2, 16
NW = NC * NS
PER_W = B // NW
CHUNK = 128
NCH = PER_W // CHUNK
L = 16
EW = E // 2
TBLK = 16384

_mesh = plsc.VectorSubcoreMesh(core_axis_name="c", subcore_axis_name="s")


@functools.partial(
    pl.kernel,
    out_type=jax.ShapeDtypeStruct((B // CHUNK, C, CHUNK), jnp.float32),
    mesh=_mesh,
    scratch_types=[
        pltpu.VMEM((CHUNK,), jnp.int32),
        pltpu.VMEM((C * CHUNK,), jnp.int32),
        pltpu.VMEM((C * CHUNK,), jnp.int32),
        pltpu.VMEM((CHUNK, EW), jnp.int32),
        pltpu.VMEM((CHUNK * C, 128), jnp.float32),
        pltpu.VMEM((C, CHUNK), jnp.float32),
        pltpu.SemaphoreType.DMA,
    ],
    compiler_params=pltpu.CompilerParams(
        needs_layout_passes=False, use_tc_tiling_on_sc=False),
)
def _w2v(t_hbm, cidx_hbm, wt_hbm, wc_hbm, out_hbm,
         t_idx_v, c_idx_v, c2_v, wt_v, wc_v, out_v, sem):
    wid = lax.axis_index("s") * NC + lax.axis_index("c")
    lanes = lax.iota(jnp.int32, L)
    zero = jnp.full((L,), 0, jnp.int32)
    for j in range(NCH):
        b0 = wid * PER_W + j * CHUNK
        n = b0 // CHUNK
        pltpu.sync_copy(t_hbm.at[pl.ds(b0, CHUNK)], t_idx_v)
        pltpu.sync_copy(cidx_hbm.at[n], c_idx_v)
        for k in range(C * CHUNK // L):
            c2_v[pl.ds(L * k, L)] = lax.shift_right_logical(
                c_idx_v[pl.ds(L * k, L)], 1)
        cps = [pltpu.async_copy(wt_hbm.at[t_idx_v], wt_v, sem)]
        for r in range(C):
            cps.append(pltpu.async_copy(
                wc_hbm.at[c2_v.at[pl.ds(r * CHUNK, CHUNK)]],
                wc_v.at[pl.ds(r * CHUNK, CHUNK)], sem))
        for cp in cps:
            cp.wait()

        for g in range(CHUNK // L):

            def bbody(i, res, g=g):
                b = g * L + i
                w = []
                for k in range(2):
                    pk = plsc.bitcast(wt_v[b, pl.ds(L * k, L)], jnp.bfloat16)
                    w.extend(plsc.unpack(pk, format=plsc.PackFormat.INTERLEAVED))
                m = lanes == i
                new = []
                for c in range(C):
                    r = b * C + c
                    cc = plsc.load_gather(c_idx_v, [zero + (b * C + c)])
                    pm = (cc & 1) == 1
                    acc = None
                    for k in range(E // L):
                        wck = jnp.where(
                            pm,
                            wc_v[r, pl.ds(64 + 16 * k, L)],
                            wc_v[r, pl.ds(16 * k, L)])
                        t = w[k] * wck
                        acc = t if acc is None else acc + t
                    new.append(jnp.where(m, jnp.sum(acc), res[c]))
                return tuple(new)

            res = lax.fori_loop(
                0, L, bbody,
                tuple(jnp.zeros((L,), jnp.float32) for _ in range(C)))
            for c in range(C):
                out_v[c, pl.ds(g * L, L)] = res[c]

        pltpu.sync_copy(out_v, out_hbm.at[n])


def _sel(half):
    src = (lax.broadcasted_iota(jnp.int32, (E, EW), 1)
           // 16 * 32 % E
           + lax.broadcasted_iota(jnp.int32, (E, EW), 1) % 16
           + half * 16)
    return (lax.broadcasted_iota(jnp.int32, (E, EW), 0) == src
            ).astype(jnp.bfloat16)


def _bf16_bits(y):
    yi = lax.bitcast_convert_type(y, jnp.int32)
    r = yi + 0x7FFF + ((yi >> 16) & 1)
    return (r >> 16) & 0xFFFF


def _tc_relayout_body(x_ref, o_ref):
    xb = x_ref[...].astype(jnp.bfloat16)
    y_lo = lax.dot_general(
        xb, _sel(0), (((0,), (0,)), ((), ())),
        preferred_element_type=jnp.float32)
    y_hi = lax.dot_general(
        xb, _sel(1), (((0,), (0,)), ((), ())),
        preferred_element_type=jnp.float32)
    o_ref[...] = _bf16_bits(y_lo) | (_bf16_bits(y_hi) << 16)


def _tc_relayout(pt):
    return pl.pallas_call(
        _tc_relayout_body,
        grid=(pl.cdiv(V, TBLK),),
        in_specs=[pl.BlockSpec((E, TBLK), lambda i: (0, i))],
        out_specs=pl.BlockSpec((TBLK, EW), lambda i: (i, 0)),
        out_shape=jax.ShapeDtypeStruct((V, EW), jnp.int32),
    )(pt)


def kernel(target, context, W_target, W_context):
    wt_packed = _tc_relayout(W_target.T)
    wc_pairs = W_context.reshape(V // 2, 128)
    cidx = context.reshape(B // CHUNK, C * CHUNK)
    out = _w2v(target, cidx, wt_packed, wc_pairs)
    return out.transpose(0, 2, 1).reshape(B, C)


# speedup vs baseline: 1.0972x; 1.0972x over previous
import functools

import jax
import jax.numpy as jnp
from jax import lax
from jax.experimental import pallas as pl
from jax.experimental.pallas import tpu as pltpu
from jax.experimental.pallas import tpu_sc as plsc

B = 16384
C = 5
E = 64
NC, NS = 2, 16
NW = NC * NS
PER_W = B // NW
CHUNK = 128
NCH = PER_W // CHUNK
L = 16

_mesh = plsc.VectorSubcoreMesh(core_axis_name="c", subcore_axis_name="s")


@functools.partial(
    pl.kernel,
    out_type=jax.ShapeDtypeStruct((B // CHUNK, C, CHUNK), jnp.float32),
    mesh=_mesh,
    scratch_types=[
        pltpu.VMEM((CHUNK,), jnp.int32),
        pltpu.VMEM((C, CHUNK), jnp.int32),
        pltpu.VMEM((CHUNK, E), jnp.float32),
        pltpu.VMEM((CHUNK * C, E), jnp.float32),
        pltpu.VMEM((C, CHUNK), jnp.float32),
        pltpu.SemaphoreType.DMA,
    ],
    compiler_params=pltpu.CompilerParams(
        needs_layout_passes=False, use_tc_tiling_on_sc=False),
)
def _w2v(t_hbm, cidx_hbm, wt_hbm, wc_hbm, out_hbm,
         t_idx_v, c_idx_v, wt_v, wc_v, out_v, sem):
    wid = lax.axis_index("s") * NC + lax.axis_index("c")
    lanes = lax.iota(jnp.int32, L)
    for j in range(NCH):
        b0 = wid * PER_W + j * CHUNK
        n = b0 // CHUNK
        pltpu.sync_copy(t_hbm.at[pl.ds(b0, CHUNK)], t_idx_v)
        pltpu.sync_copy(cidx_hbm.at[n], c_idx_v)
        cps = [pltpu.async_copy(wt_hbm.at[t_idx_v], wt_v, sem)]
        for r in range(C):
            cps.append(pltpu.async_copy(
                wc_hbm.at[c_idx_v.at[r]],
                wc_v.at[pl.ds(r * CHUNK, CHUNK)], sem))
        for cp in cps:
            cp.wait()

        for g in range(CHUNK // L):

            def bbody(i, res, g=g):
                b = g * L + i
                w = [wt_v[b, pl.ds(16 * k, L)] for k in range(E // L)]
                m = lanes == i
                new = []
                for c in range(C):
                    r = b * C + c
                    acc = w[0] * wc_v[r, pl.ds(0, L)]
                    for k in range(1, E // L):
                        acc = acc + w[k] * wc_v[r, pl.ds(16 * k, L)]
                    new.append(jnp.where(m, jnp.sum(acc), res[c]))
                return tuple(new)

            res = lax.fori_loop(
                0, L, bbody,
                tuple(jnp.zeros((L,), jnp.float32) for _ in range(C)))
            for c in range(C):
                out_v[c, pl.ds(g * L, L)] = res[c]

        pltpu.sync_copy(out_v, out_hbm.at[n])


def kernel(target, context, W_target, W_context):
    cidx = context.reshape(-1).reshape(B // CHUNK, C, CHUNK)
    out = _w2v(target, cidx, W_target, W_context)
    return out.transpose(0, 2, 1).reshape(B, C)
